# baseline (speedup 1.0000x reference)
import functools

import jax
import jax.numpy as jnp
from jax import lax
from jax.experimental import pallas as pl
from jax.experimental.pallas import tpu as pltpu
from jax.experimental.pallas import tpu_sc as plsc

NC = 2
NS = 16
NW = NC * NS
L = 16

_HI = jax.lax.Precision.HIGHEST
_SC_PARAMS = pltpu.CompilerParams(needs_layout_passes=False,
                                  use_tc_tiling_on_sc=False)


def _fl(x):
    return jnp.full((L,), x, jnp.int32)


def _proj_body(h_ref, wq_ref, bq_ref, wk_ref, bk_ref, wv_ref, bv_ref,
               wmsg_ref, bmsg_ref, wattn_ref, battn_ref, kv_ref, q_ref):
    h = h_ref[...]
    dn = (((1,), (1,)), ((), ()))
    wkc = jax.lax.dot_general(wattn_ref[...], wk_ref[...],
                              (((1,), (0,)), ((), ())), precision=_HI)
    bkc = jax.lax.dot_general(bk_ref[...], wattn_ref[...], dn,
                              precision=_HI) + battn_ref[...]
    wvc = jax.lax.dot_general(wmsg_ref[...], wv_ref[...],
                              (((1,), (0,)), ((), ())), precision=_HI)
    bvc = jax.lax.dot_general(bv_ref[...], wmsg_ref[...], dn,
                              precision=_HI) + bmsg_ref[...]
    k = jax.lax.dot_general(h, wkc, dn, precision=_HI) + bkc
    v = jax.lax.dot_general(h, wvc, dn, precision=_HI) + bvc
    q = jax.lax.dot_general(h, wq_ref[...], dn, precision=_HI) + bq_ref[...]
    kv_ref[:, :128] = k
    kv_ref[:, 128:] = v
    q_ref[...] = q


def _project(h, Wq, bq, Wk, bk, Wv, bv, Wmsg, bmsg, Wattn, battn):
    n = h.shape[0]
    return pl.pallas_call(
        _proj_body,
        out_shape=[jax.ShapeDtypeStruct((n, 256), jnp.float32),
                   jax.ShapeDtypeStruct((n, 128), jnp.float32)],
    )(h, Wq, bq, Wk, bk, Wv, bv, Wmsg, bmsg, Wattn, battn)


def _edge_pass(kv, q, src, dst):
    n = kv.shape[0]
    e = src.shape[0]
    C = 32
    nchunks = e // C
    nb, rem = divmod(nchunks, NW)
    npad = (n + NS * C - 1) // (NS * C) * (NS * C)
    rpt = npad // NS
    assert nchunks * C == e and rpt % C == 0 and C % L == 0 and nb >= 3

    mesh = plsc.VectorSubcoreMesh(core_axis_name="c", subcore_axis_name="s")

    @functools.partial(
        pl.kernel,
        out_type=jax.ShapeDtypeStruct((NC, npad, 144), jnp.float32),
        mesh=mesh,
        scratch_types=[
            pltpu.VMEM((4, C), jnp.int32),
            pltpu.VMEM((4, C), jnp.int32),
            pltpu.VMEM((2, C, 256), jnp.float32),
            pltpu.VMEM((2, C, 128), jnp.float32),
            pltpu.VMEM((2, C, 144), jnp.float32),
            pltpu.VMEM_SHARED((npad, 144), jnp.float32),
            pltpu.SemaphoreType.DMA((4,)),
            pltpu.SemaphoreType.DMA((4,)),
            pltpu.SemaphoreType.DMA((2,)),
            pltpu.SemaphoreType.DMA((2,)),
            pltpu.SemaphoreType.DMA((2,)),
        ],
        compiler_params=_SC_PARAMS,
    )
    def ek(kv_hbm, q_hbm, src_hbm, dst_hbm, out_hbm,
           sidx, didx, kvb, qb, wb, acc_sh, si, sd, skv, sq, ss):
        cid = lax.axis_index("c")
        sid = lax.axis_index("s")
        wid = sid * NC + cid

        zv = jnp.zeros((L,), jnp.float32)

        for pz in range(2):
            @pl.loop(0, C)
            def _(r):
                @pl.loop(0, 9)
                def _(cc):
                    wb.at[pz, r, pl.ds(cc * 16, 16)][...] = zv

        @pl.loop(0, rpt // C)
        def _(i):
            pltpu.sync_copy(wb.at[0], acc_sh.at[pl.ds(sid * rpt + i * C, C)])

        plsc.subcore_barrier()

        nchw = nb + jnp.where(wid < rem, 1, 0)
        rows0 = jax.lax.iota(jnp.int32, L)

        def base_of(ch):
            return (wid + ch * NW) * C

        def issue_idx(ch):
            r = lax.rem(ch, 4)
            pltpu.async_copy(src_hbm.at[pl.ds(base_of(ch), C)], sidx.at[r],
                             si.at[r])
            pltpu.async_copy(dst_hbm.at[pl.ds(base_of(ch), C)], didx.at[r],
                             sd.at[r])

        def wait_idx(ch):
            r = lax.rem(ch, 4)
            pltpu.make_async_copy(src_hbm.at[pl.ds(base_of(ch), C)],
                                  sidx.at[r], si.at[r]).wait()
            pltpu.make_async_copy(dst_hbm.at[pl.ds(base_of(ch), C)],
                                  didx.at[r], sd.at[r]).wait()

        def issue_gather(ch):
            r = lax.rem(ch, 4)
            p = lax.rem(ch, 2)
            pltpu.async_copy(kv_hbm.at[sidx.at[r]], kvb.at[p], skv.at[p])
            pltpu.async_copy(q_hbm.at[didx.at[r]], qb.at[p], sq.at[p])

        def wait_gather(ch):
            r = lax.rem(ch, 4)
            p = lax.rem(ch, 2)
            pltpu.make_async_copy(kv_hbm.at[sidx.at[r]], kvb.at[p],
                                  skv.at[p]).wait()
            pltpu.make_async_copy(q_hbm.at[didx.at[r]], qb.at[p],
                                  sq.at[p]).wait()

        def issue_scatter(ch):
            r = lax.rem(ch, 4)
            p = lax.rem(ch, 2)
            pltpu.async_copy(wb.at[p], acc_sh.at[didx.at[r]], ss.at[p],
                             add=True)

        def wait_scatter(ch):
            r = lax.rem(ch, 4)
            p = lax.rem(ch, 2)
            pltpu.make_async_copy(wb.at[p], acc_sh.at[didx.at[r]],
                                  ss.at[p]).wait()

        def compute(ch):
            p = lax.rem(ch, 2)
            kvp = kvb.at[p]
            qp = qb.at[p]
            wp = wb.at[p]
            for g in range(C // L):
                rows = rows0 + (g * L)
                for hh in range(8):
                    acc = jnp.zeros((L,), jnp.float32)
                    for j in range(L):
                        col = hh * 16 + j
                        acc = acc + (plsc.load_gather(kvp, [rows, _fl(col)])
                                     * plsc.load_gather(qp, [rows, _fl(col)]))
                    ev = jnp.exp(acc * 0.25)
                    plsc.store_scatter(wp, [rows, _fl(128 + hh)], ev)
                    for j in range(L):
                        vv = plsc.load_gather(kvp,
                                              [rows, _fl(128 + hh * 16 + j)])
                        plsc.store_scatter(wp, [rows, _fl(hh * 16 + j)],
                                           ev * vv)

        issue_idx(0)
        issue_idx(1)
        wait_idx(0)
        issue_gather(0)

        @pl.loop(0, nchw)
        def _(i):
            @pl.when(i + 1 < nchw)
            def _():
                wait_idx(i + 1)
                issue_gather(i + 1)
            wait_gather(i)

            @pl.when(i >= 2)
            def _():
                wait_scatter(i - 2)

            @pl.when(i + 2 < nchw)
            def _():
                issue_idx(i + 2)
            compute(i)
            issue_scatter(i)

        wait_scatter(nchw - 2)
        wait_scatter(nchw - 1)

        plsc.subcore_barrier()
        pltpu.sync_copy(acc_sh.at[pl.ds(sid * rpt, rpt)],
                        out_hbm.at[cid, pl.ds(sid * rpt, rpt)])

    return ek(kv, q, src, dst)


def _fin_body(ue_ref, wa_ref, ba_ref, hout_ref):
    n = hout_ref.shape[0]
    ue = ue_ref[0, :n] + ue_ref[1, :n]
    u = ue[:, :128]
    s = ue[:, 128:136]
    col = jax.lax.broadcasted_iota(jnp.int32, (8, 128), 1)
    row = jax.lax.broadcasted_iota(jnp.int32, (8, 128), 0)
    ind = jnp.where(col // 16 == row, 1.0, 0.0).astype(jnp.float32)
    s_exp = jax.lax.dot_general(s, ind, (((1,), (0,)), ((), ())), precision=_HI)
    agg = u / jnp.maximum(s_exp, 1e-30)
    hout = jax.lax.dot_general(agg, wa_ref[...], (((1,), (1,)), ((), ())),
                               precision=_HI) + ba_ref[...]
    hout_ref[...] = hout


def _finalize(ue, Wa, ba, n):
    return pl.pallas_call(
        _fin_body,
        out_shape=jax.ShapeDtypeStruct((n, 128), jnp.float32),
    )(ue, Wa, ba)


def _score_pass(hout, asrc, adst):
    te = asrc.shape[0]
    C = 64
    nchunks = te // C
    nb, rem = divmod(nchunks, NW)
    assert nchunks * C == te and C % L == 0 and nb >= 3

    mesh = plsc.VectorSubcoreMesh(core_axis_name="c", subcore_axis_name="s")

    @functools.partial(
        pl.kernel,
        out_type=jax.ShapeDtypeStruct((te,), jnp.float32),
        mesh=mesh,
        scratch_types=[
            pltpu.VMEM((4, C), jnp.int32),
            pltpu.VMEM((4, C), jnp.int32),
            pltpu.VMEM((2, C, 256), jnp.float32),
            pltpu.VMEM((2, C, 256), jnp.float32),
            pltpu.VMEM((2, C), jnp.float32),
            pltpu.SemaphoreType.DMA((4,)),
            pltpu.SemaphoreType.DMA((4,)),
            pltpu.SemaphoreType.DMA((2,)),
            pltpu.SemaphoreType.DMA((2,)),
            pltpu.SemaphoreType.DMA((2,)),
        ],
        compiler_params=_SC_PARAMS,
    )
    def sk(h_hbm, src_hbm, dst_hbm, out_hbm,
           sidx, didx, ab, bb, ob, si, sd, sa, sb, so):
        cid = lax.axis_index("c")
        sid = lax.axis_index("s")
        wid = sid * NC + cid
        nchw = nb + jnp.where(wid < rem, 1, 0)
        rows0 = jax.lax.iota(jnp.int32, L)

        def base_of(ch):
            return (wid + ch * NW) * C

        def issue_idx(ch):
            r = lax.rem(ch, 4)
            pltpu.async_copy(src_hbm.at[pl.ds(base_of(ch), C)], sidx.at[r],
                             si.at[r])
            pltpu.async_copy(dst_hbm.at[pl.ds(base_of(ch), C)], didx.at[r],
                             sd.at[r])

        def wait_idx(ch):
            r = lax.rem(ch, 4)
            pltpu.make_async_copy(src_hbm.at[pl.ds(base_of(ch), C)],
                                  sidx.at[r], si.at[r]).wait()
            pltpu.make_async_copy(dst_hbm.at[pl.ds(base_of(ch), C)],
                                  didx.at[r], sd.at[r]).wait()

        def issue_gather(ch):
            r = lax.rem(ch, 4)
            p = lax.rem(ch, 2)
            pltpu.async_copy(h_hbm.at[sidx.at[r]], ab.at[p], sa.at[p])
            pltpu.async_copy(h_hbm.at[didx.at[r]], bb.at[p], sb.at[p])

        def wait_gather(ch):
            r = lax.rem(ch, 4)
            p = lax.rem(ch, 2)
            pltpu.make_async_copy(h_hbm.at[sidx.at[r]], ab.at[p],
                                  sa.at[p]).wait()
            pltpu.make_async_copy(h_hbm.at[didx.at[r]], bb.at[p],
                                  sb.at[p]).wait()

        def issue_out(ch):
            p = lax.rem(ch, 2)
            pltpu.async_copy(ob.at[p], out_hbm.at[pl.ds(base_of(ch), C)],
                             so.at[p])

        def wait_out(ch):
            p = lax.rem(ch, 2)
            pltpu.make_async_copy(ob.at[p], out_hbm.at[pl.ds(base_of(ch), C)],
                                  so.at[p]).wait()

        def compute(ch):
            p = lax.rem(ch, 2)
            ap = ab.at[p]
            bp = bb.at[p]
            op = ob.at[p]
            for g in range(C // L):
                rows = rows0 + (g * L)
                acc = jnp.zeros((L,), jnp.float32)
                for col in range(128):
                    acc = acc + (plsc.load_gather(ap, [rows, _fl(col)])
                                 * plsc.load_gather(bp, [rows, _fl(col)]))
                op.at[pl.ds(g * L, L)][...] = acc

        issue_idx(0)
        issue_idx(1)
        wait_idx(0)
        issue_gather(0)

        @pl.loop(0, nchw)
        def _(i):
            @pl.when(i + 1 < nchw)
            def _():
                wait_idx(i + 1)
                issue_gather(i + 1)
            wait_gather(i)

            @pl.when(i >= 2)
            def _():
                wait_out(i - 2)

            @pl.when(i + 2 < nchw)
            def _():
                issue_idx(i + 2)
            compute(i)
            issue_out(i)

        wait_out(nchw - 2)
        wait_out(nchw - 1)

    return sk(hout, asrc, adst)


def kernel(h, edge_index, neg_edge_index, Wq, bq, Wk, bk, Wv, bv,
           Wmsg, bmsg, Wattn, battn, Wa, ba):
    e = edge_index.shape[1]
    src = edge_index[0].astype(jnp.int32)
    dst = edge_index[1].astype(jnp.int32)
    nsrc = neg_edge_index[0].astype(jnp.int32)
    ndst = neg_edge_index[1].astype(jnp.int32)

    kv, q = _project(h, Wq, bq.reshape(1, -1), Wk, bk.reshape(1, -1),
                     Wv, bv.reshape(1, -1), Wmsg, bmsg.reshape(1, -1),
                     Wattn, battn.reshape(1, -1))
    ue = _edge_pass(kv, q, src, dst)
    hout = _finalize(ue, Wa, ba.reshape(1, -1), h.shape[0])

    asrc = jnp.concatenate([src, nsrc])
    adst = jnp.concatenate([dst, ndst])
    sc = _score_pass(kv, asrc, adst)
    score = sc[:e].reshape(e, 1, 1)
    neg_score = sc[e:].reshape(e, 1, 1)
    return hout[:, None, :], score, neg_score


# speedup vs baseline: 1.0028x; 1.0028x over previous
import functools

import jax
import jax.numpy as jnp
from jax import lax
from jax.experimental import pallas as pl
from jax.experimental.pallas import tpu as pltpu
from jax.experimental.pallas import tpu_sc as plsc

NC = 2
NS = 16
NW = NC * NS
L = 16

_HI = jax.lax.Precision.HIGHEST
_SC_PARAMS = pltpu.CompilerParams(needs_layout_passes=False,
                                  use_tc_tiling_on_sc=False)


def _fl(x):
    return jnp.full((L,), x, jnp.int32)


def _proj_body(h_ref, wq_ref, bq_ref, wk_ref, bk_ref, wv_ref, bv_ref,
               wmsg_ref, bmsg_ref, wattn_ref, battn_ref, kv_ref, q_ref):
    h = h_ref[...]
    dn = (((1,), (1,)), ((), ()))
    wkc = jax.lax.dot_general(wattn_ref[...], wk_ref[...],
                              (((1,), (0,)), ((), ())), precision=_HI)
    bkc = jax.lax.dot_general(bk_ref[...], wattn_ref[...], dn,
                              precision=_HI) + battn_ref[...]
    wvc = jax.lax.dot_general(wmsg_ref[...], wv_ref[...],
                              (((1,), (0,)), ((), ())), precision=_HI)
    bvc = jax.lax.dot_general(bv_ref[...], wmsg_ref[...], dn,
                              precision=_HI) + bmsg_ref[...]
    k = jax.lax.dot_general(h, wkc, dn, precision=_HI) + bkc
    v = jax.lax.dot_general(h, wvc, dn, precision=_HI) + bvc
    q = jax.lax.dot_general(h, wq_ref[...], dn, precision=_HI) + bq_ref[...]
    kv_ref[:, :128] = k
    kv_ref[:, 128:] = v
    q_ref[...] = q


def _project(h, Wq, bq, Wk, bk, Wv, bv, Wmsg, bmsg, Wattn, battn):
    n = h.shape[0]
    return pl.pallas_call(
        _proj_body,
        out_shape=[jax.ShapeDtypeStruct((n, 256), jnp.float32),
                   jax.ShapeDtypeStruct((n, 128), jnp.float32)],
    )(h, Wq, bq, Wk, bk, Wv, bv, Wmsg, bmsg, Wattn, battn)


def _edge_pass(kv, q, src, dst):
    n = kv.shape[0]
    e = src.shape[0]
    C = 32
    nchunks = e // C
    nb, rem = divmod(nchunks, NW)
    npad = (n + NS * C - 1) // (NS * C) * (NS * C)
    rpt = npad // NS
    assert nchunks * C == e and rpt % C == 0 and C % L == 0 and nb >= 3

    mesh = plsc.VectorSubcoreMesh(core_axis_name="c", subcore_axis_name="s")

    @functools.partial(
        pl.kernel,
        out_type=jax.ShapeDtypeStruct((NC, npad, 144), jnp.float32),
        mesh=mesh,
        scratch_types=[
            pltpu.VMEM((4, C), jnp.int32),
            pltpu.VMEM((4, C), jnp.int32),
            pltpu.VMEM((2, C, 256), jnp.float32),
            pltpu.VMEM((2, C, 128), jnp.float32),
            pltpu.VMEM((2, C, 144), jnp.float32),
            pltpu.VMEM_SHARED((npad, 144), jnp.float32),
            pltpu.SemaphoreType.DMA((4,)),
            pltpu.SemaphoreType.DMA((4,)),
            pltpu.SemaphoreType.DMA((2,)),
            pltpu.SemaphoreType.DMA((2,)),
            pltpu.SemaphoreType.DMA((2,)),
        ],
        compiler_params=_SC_PARAMS,
    )
    def ek(kv_hbm, q_hbm, src_hbm, dst_hbm, out_hbm,
           sidx, didx, kvb, qb, wb, acc_sh, si, sd, skv, sq, ss):
        cid = lax.axis_index("c")
        sid = lax.axis_index("s")
        wid = sid * NC + cid

        zv = jnp.zeros((L,), jnp.float32)

        for pz in range(2):
            @pl.loop(0, C)
            def _(r):
                @pl.loop(0, 9)
                def _(cc):
                    wb.at[pz, r, pl.ds(cc * 16, 16)][...] = zv

        @pl.loop(0, rpt // C)
        def _(i):
            pltpu.sync_copy(wb.at[0], acc_sh.at[pl.ds(sid * rpt + i * C, C)])

        plsc.subcore_barrier()

        nchw = nb + jnp.where(wid < rem, 1, 0)
        rows0 = jax.lax.iota(jnp.int32, L)

        def base_of(ch):
            return (wid + ch * NW) * C

        def issue_idx(ch):
            r = lax.rem(ch, 4)
            pltpu.async_copy(src_hbm.at[pl.ds(base_of(ch), C)], sidx.at[r],
                             si.at[r])
            pltpu.async_copy(dst_hbm.at[pl.ds(base_of(ch), C)], didx.at[r],
                             sd.at[r])

        def wait_idx(ch):
            r = lax.rem(ch, 4)
            pltpu.make_async_copy(src_hbm.at[pl.ds(base_of(ch), C)],
                                  sidx.at[r], si.at[r]).wait()
            pltpu.make_async_copy(dst_hbm.at[pl.ds(base_of(ch), C)],
                                  didx.at[r], sd.at[r]).wait()

        def issue_gather(ch):
            r = lax.rem(ch, 4)
            p = lax.rem(ch, 2)
            pltpu.async_copy(kv_hbm.at[sidx.at[r]], kvb.at[p], skv.at[p])
            pltpu.async_copy(q_hbm.at[didx.at[r]], qb.at[p], sq.at[p])

        def wait_gather(ch):
            r = lax.rem(ch, 4)
            p = lax.rem(ch, 2)
            pltpu.make_async_copy(kv_hbm.at[sidx.at[r]], kvb.at[p],
                                  skv.at[p]).wait()
            pltpu.make_async_copy(q_hbm.at[didx.at[r]], qb.at[p],
                                  sq.at[p]).wait()

        def issue_scatter(ch):
            r = lax.rem(ch, 4)
            p = lax.rem(ch, 2)
            pltpu.async_copy(wb.at[p], acc_sh.at[didx.at[r]], ss.at[p],
                             add=True)

        def wait_scatter(ch):
            r = lax.rem(ch, 4)
            p = lax.rem(ch, 2)
            pltpu.make_async_copy(wb.at[p], acc_sh.at[didx.at[r]],
                                  ss.at[p]).wait()

        def compute(ch):
            p = lax.rem(ch, 2)
            kvp = kvb.at[p]
            qp = qb.at[p]
            wp = wb.at[p]
            for g in range(C // L):
                rows = rows0 + (g * L)
                for hh in range(8):
                    acc = jnp.zeros((L,), jnp.float32)
                    for j in range(L):
                        col = hh * 16 + j
                        acc = acc + (plsc.load_gather(kvp, [rows, _fl(col)])
                                     * plsc.load_gather(qp, [rows, _fl(col)]))
                    ev = jnp.exp(acc * 0.25)
                    plsc.store_scatter(wp, [rows, _fl(128 + hh)], ev)
                    for j in range(L):
                        vv = plsc.load_gather(kvp,
                                              [rows, _fl(128 + hh * 16 + j)])
                        plsc.store_scatter(wp, [rows, _fl(hh * 16 + j)],
                                           ev * vv)

        issue_idx(0)
        issue_idx(1)
        wait_idx(0)
        issue_gather(0)

        @pl.loop(0, nchw)
        def _(i):
            @pl.when(i + 1 < nchw)
            def _():
                wait_idx(i + 1)
                issue_gather(i + 1)
            wait_gather(i)

            @pl.when(i >= 2)
            def _():
                wait_scatter(i - 2)

            @pl.when(i + 2 < nchw)
            def _():
                issue_idx(i + 2)
            compute(i)
            issue_scatter(i)

        wait_scatter(nchw - 2)
        wait_scatter(nchw - 1)

        plsc.subcore_barrier()
        pltpu.sync_copy(acc_sh.at[pl.ds(sid * rpt, rpt)],
                        out_hbm.at[cid, pl.ds(sid * rpt, rpt)])

    return ek(kv, q, src, dst)


def _fin_body(ue_ref, wa_ref, ba_ref, hout_ref):
    n = hout_ref.shape[0]
    ue = ue_ref[0, :n] + ue_ref[1, :n]
    u = ue[:, :128]
    s = ue[:, 128:136]
    col = jax.lax.broadcasted_iota(jnp.int32, (8, 128), 1)
    row = jax.lax.broadcasted_iota(jnp.int32, (8, 128), 0)
    ind = jnp.where(col // 16 == row, 1.0, 0.0).astype(jnp.float32)
    s_exp = jax.lax.dot_general(s, ind, (((1,), (0,)), ((), ())), precision=_HI)
    agg = u / jnp.maximum(s_exp, 1e-30)
    hout = jax.lax.dot_general(agg, wa_ref[...], (((1,), (1,)), ((), ())),
                               precision=_HI) + ba_ref[...]
    hout_ref[...] = hout


def _finalize(ue, Wa, ba, n):
    return pl.pallas_call(
        _fin_body,
        out_shape=jax.ShapeDtypeStruct((n, 128), jnp.float32),
    )(ue, Wa, ba)


def _score_pass(hout, asrc, adst):
    te = asrc.shape[0]
    C = 64
    nchunks = te // C
    nb, rem = divmod(nchunks, NW)
    assert nchunks * C == te and C % L == 0 and nb >= 8

    mesh = plsc.VectorSubcoreMesh(core_axis_name="c", subcore_axis_name="s")

    @functools.partial(
        pl.kernel,
        out_type=jax.ShapeDtypeStruct((te,), jnp.float32),
        mesh=mesh,
        scratch_types=[
            pltpu.VMEM((8, C), jnp.int32),
            pltpu.VMEM((8, C), jnp.int32),
            pltpu.VMEM((4, C, 128), jnp.float32),
            pltpu.VMEM((4, C, 128), jnp.float32),
            pltpu.VMEM((4, C), jnp.float32),
            pltpu.SemaphoreType.DMA((8,)),
            pltpu.SemaphoreType.DMA((8,)),
            pltpu.SemaphoreType.DMA((4,)),
            pltpu.SemaphoreType.DMA((4,)),
            pltpu.SemaphoreType.DMA((4,)),
        ],
        compiler_params=_SC_PARAMS,
    )
    def sk(h_hbm, src_hbm, dst_hbm, out_hbm,
           sidx, didx, ab, bb, ob, si, sd, sa, sb, so):
        cid = lax.axis_index("c")
        sid = lax.axis_index("s")
        wid = sid * NC + cid
        nchw = nb + jnp.where(wid < rem, 1, 0)
        rows0 = jax.lax.iota(jnp.int32, L)

        def base_of(ch):
            return (wid + ch * NW) * C

        def issue_idx(ch):
            r = lax.rem(ch, 8)
            pltpu.async_copy(src_hbm.at[pl.ds(base_of(ch), C)], sidx.at[r],
                             si.at[r])
            pltpu.async_copy(dst_hbm.at[pl.ds(base_of(ch), C)], didx.at[r],
                             sd.at[r])

        def wait_idx(ch):
            r = lax.rem(ch, 8)
            pltpu.make_async_copy(src_hbm.at[pl.ds(base_of(ch), C)],
                                  sidx.at[r], si.at[r]).wait()
            pltpu.make_async_copy(dst_hbm.at[pl.ds(base_of(ch), C)],
                                  didx.at[r], sd.at[r]).wait()

        def issue_gather(ch):
            r = lax.rem(ch, 8)
            p = lax.rem(ch, 4)
            pltpu.async_copy(h_hbm.at[sidx.at[r]], ab.at[p], sa.at[p])
            pltpu.async_copy(h_hbm.at[didx.at[r]], bb.at[p], sb.at[p])

        def wait_gather(ch):
            r = lax.rem(ch, 8)
            p = lax.rem(ch, 4)
            pltpu.make_async_copy(h_hbm.at[sidx.at[r]], ab.at[p],
                                  sa.at[p]).wait()
            pltpu.make_async_copy(h_hbm.at[didx.at[r]], bb.at[p],
                                  sb.at[p]).wait()

        def issue_out(ch):
            p = lax.rem(ch, 4)
            pltpu.async_copy(ob.at[p], out_hbm.at[pl.ds(base_of(ch), C)],
                             so.at[p])

        def wait_out(ch):
            p = lax.rem(ch, 4)
            pltpu.make_async_copy(ob.at[p], out_hbm.at[pl.ds(base_of(ch), C)],
                                  so.at[p]).wait()

        def compute(ch):
            p = lax.rem(ch, 4)
            ap = ab.at[p]
            bp = bb.at[p]
            op = ob.at[p]
            for g in range(C // L):
                rows = rows0 + (g * L)
                acc = jnp.zeros((L,), jnp.float32)
                for col in range(128):
                    acc = acc + (plsc.load_gather(ap, [rows, _fl(col)])
                                 * plsc.load_gather(bp, [rows, _fl(col)]))
                op.at[pl.ds(g * L, L)][...] = acc

        for k in range(5):
            issue_idx(k)
        for k in range(3):
            wait_idx(k)
            issue_gather(k)

        @pl.loop(0, nchw)
        def _(i):
            @pl.when(i + 3 < nchw)
            def _():
                wait_idx(i + 3)
                issue_gather(i + 3)
            wait_gather(i)

            @pl.when(i >= 3)
            def _():
                wait_out(i - 3)

            @pl.when(i + 5 < nchw)
            def _():
                issue_idx(i + 5)
            compute(i)
            issue_out(i)

        wait_out(nchw - 3)
        wait_out(nchw - 2)
        wait_out(nchw - 1)

    return sk(hout, asrc, adst)


def kernel(h, edge_index, neg_edge_index, Wq, bq, Wk, bk, Wv, bv,
           Wmsg, bmsg, Wattn, battn, Wa, ba):
    e = edge_index.shape[1]
    src = edge_index[0].astype(jnp.int32)
    dst = edge_index[1].astype(jnp.int32)
    nsrc = neg_edge_index[0].astype(jnp.int32)
    ndst = neg_edge_index[1].astype(jnp.int32)

    kv, q = _project(h, Wq, bq.reshape(1, -1), Wk, bk.reshape(1, -1),
                     Wv, bv.reshape(1, -1), Wmsg, bmsg.reshape(1, -1),
                     Wattn, battn.reshape(1, -1))
    ue = _edge_pass(kv, q, src, dst)
    hout = _finalize(ue, Wa, ba.reshape(1, -1), h.shape[0])

    asrc = jnp.concatenate([src, nsrc])
    adst = jnp.concatenate([dst, ndst])
    sc = _score_pass(hout, asrc, adst)
    score = sc[:e].reshape(e, 1, 1)
    neg_score = sc[e:].reshape(e, 1, 1)
    return hout[:, None, :], score, neg_score


# speedup vs baseline: 2.9706x; 2.9623x over previous
import functools

import jax
import jax.numpy as jnp
from jax import lax
from jax.experimental import pallas as pl
from jax.experimental.pallas import tpu as pltpu
from jax.experimental.pallas import tpu_sc as plsc

NC = 2
NS = 16
NW = NC * NS
L = 16

_HI = jax.lax.Precision.HIGHEST
_SC_PARAMS = pltpu.CompilerParams(needs_layout_passes=False,
                                  use_tc_tiling_on_sc=False)


def _fl(x):
    return jnp.full((L,), x, jnp.int32)


def _proj_body(h_ref, wq_ref, bq_ref, wk_ref, bk_ref, wv_ref, bv_ref,
               wmsg_ref, bmsg_ref, wattn_ref, battn_ref, kv_ref, q_ref):
    h = h_ref[...]
    dn = (((1,), (1,)), ((), ()))
    wkc = jax.lax.dot_general(wattn_ref[...], wk_ref[...],
                              (((1,), (0,)), ((), ())), precision=_HI)
    bkc = jax.lax.dot_general(bk_ref[...], wattn_ref[...], dn,
                              precision=_HI) + battn_ref[...]
    wvc = jax.lax.dot_general(wmsg_ref[...], wv_ref[...],
                              (((1,), (0,)), ((), ())), precision=_HI)
    bvc = jax.lax.dot_general(bv_ref[...], wmsg_ref[...], dn,
                              precision=_HI) + bmsg_ref[...]
    k = jax.lax.dot_general(h, wkc, dn, precision=_HI) + bkc
    v = jax.lax.dot_general(h, wvc, dn, precision=_HI) + bvc
    q = jax.lax.dot_general(h, wq_ref[...], dn, precision=_HI) + bq_ref[...]
    ri = jax.lax.broadcasted_iota(jnp.int32, (128, 128), 0)
    ci = jax.lax.broadcasted_iota(jnp.int32, (128, 128), 1)
    perm = jnp.where(ci == (ri % 16) * 8 + ri // 16, 1.0, 0.0)
    perm = perm.astype(jnp.float32)
    pdn = (((1,), (0,)), ((), ()))
    kv_ref[:, :128] = jax.lax.dot_general(k, perm, pdn, precision=_HI)
    kv_ref[:, 128:] = v
    q_ref[...] = jax.lax.dot_general(q, perm, pdn, precision=_HI)


def _project(h, Wq, bq, Wk, bk, Wv, bv, Wmsg, bmsg, Wattn, battn):
    n = h.shape[0]
    return pl.pallas_call(
        _proj_body,
        out_shape=[jax.ShapeDtypeStruct((n, 256), jnp.float32),
                   jax.ShapeDtypeStruct((n, 128), jnp.float32)],
    )(h, Wq, bq, Wk, bk, Wv, bv, Wmsg, bmsg, Wattn, battn)


def _edge_pass(kv, q, src, dst):
    n = kv.shape[0]
    e = src.shape[0]
    C = 32
    nchunks = e // C
    nb, rem = divmod(nchunks, NW)
    npad = (n + NS * C - 1) // (NS * C) * (NS * C)
    rpt = npad // NS
    assert nchunks * C == e and rpt % C == 0 and C % L == 0 and nb >= 3

    mesh = plsc.VectorSubcoreMesh(core_axis_name="c", subcore_axis_name="s")

    @functools.partial(
        pl.kernel,
        out_type=jax.ShapeDtypeStruct((NC, npad, 144), jnp.float32),
        mesh=mesh,
        scratch_types=[
            pltpu.VMEM((4, C), jnp.int32),
            pltpu.VMEM((4, C), jnp.int32),
            pltpu.VMEM((2, C, 256), jnp.float32),
            pltpu.VMEM((2, C, 128), jnp.float32),
            pltpu.VMEM((2, C, 144), jnp.float32),
            pltpu.VMEM((C, 32), jnp.float32),
            pltpu.VMEM_SHARED((npad, 144), jnp.float32),
            pltpu.SemaphoreType.DMA((4,)),
            pltpu.SemaphoreType.DMA((4,)),
            pltpu.SemaphoreType.DMA((2,)),
            pltpu.SemaphoreType.DMA((2,)),
            pltpu.SemaphoreType.DMA((2,)),
        ],
        compiler_params=_SC_PARAMS,
    )
    def ek(kv_hbm, q_hbm, src_hbm, dst_hbm, out_hbm,
           sidx, didx, kvb, qb, wb, rot, acc_sh, si, sd, skv, sq, ss):
        cid = lax.axis_index("c")
        sid = lax.axis_index("s")
        wid = sid * NC + cid

        zv = jnp.zeros((L,), jnp.float32)

        for pz in range(2):
            @pl.loop(0, C)
            def _(r):
                @pl.loop(0, 9)
                def _(cc):
                    wb.at[pz, r, pl.ds(cc * 16, 16)][...] = zv

        @pl.loop(0, rpt // C)
        def _(i):
            pltpu.sync_copy(wb.at[0], acc_sh.at[pl.ds(sid * rpt + i * C, C)])

        plsc.subcore_barrier()

        nchw = nb + jnp.where(wid < rem, 1, 0)
        rows0 = jax.lax.iota(jnp.int32, L)

        def base_of(ch):
            return (wid + ch * NW) * C

        def issue_idx(ch):
            r = lax.rem(ch, 4)
            pltpu.async_copy(src_hbm.at[pl.ds(base_of(ch), C)], sidx.at[r],
                             si.at[r])
            pltpu.async_copy(dst_hbm.at[pl.ds(base_of(ch), C)], didx.at[r],
                             sd.at[r])

        def wait_idx(ch):
            r = lax.rem(ch, 4)
            pltpu.make_async_copy(src_hbm.at[pl.ds(base_of(ch), C)],
                                  sidx.at[r], si.at[r]).wait()
            pltpu.make_async_copy(dst_hbm.at[pl.ds(base_of(ch), C)],
                                  didx.at[r], sd.at[r]).wait()

        def issue_gather(ch):
            r = lax.rem(ch, 4)
            p = lax.rem(ch, 2)
            pltpu.async_copy(kv_hbm.at[sidx.at[r]], kvb.at[p], skv.at[p])
            pltpu.async_copy(q_hbm.at[didx.at[r]], qb.at[p], sq.at[p])

        def wait_gather(ch):
            r = lax.rem(ch, 4)
            p = lax.rem(ch, 2)
            pltpu.make_async_copy(kv_hbm.at[sidx.at[r]], kvb.at[p],
                                  skv.at[p]).wait()
            pltpu.make_async_copy(q_hbm.at[didx.at[r]], qb.at[p],
                                  sq.at[p]).wait()

        def issue_scatter(ch):
            r = lax.rem(ch, 4)
            p = lax.rem(ch, 2)
            pltpu.async_copy(wb.at[p], acc_sh.at[didx.at[r]], ss.at[p],
                             add=True)

        def wait_scatter(ch):
            r = lax.rem(ch, 4)
            p = lax.rem(ch, 2)
            pltpu.make_async_copy(wb.at[p], acc_sh.at[didx.at[r]],
                                  ss.at[p]).wait()

        def compute(ch):
            p = lax.rem(ch, 2)
            kvp = kvb.at[p]
            qp = qb.at[p]
            wp = wb.at[p]
            for r in range(C):
                acc = None
                for m in range(8):
                    prod = (kvp[r, pl.ds(m * 16, 16)]
                            * qp[r, pl.ds(m * 16, 16)])
                    acc = prod if acc is None else acc + prod
                rot.at[r, pl.ds(0, 16)][...] = acc
                rot.at[r, pl.ds(16, 16)][...] = acc
                accs = acc + rot[r, pl.ds(8, 16)]
                ev = jnp.exp(accs * 0.25)
                wp.at[r, pl.ds(128, 16)][...] = ev
                for m in range(8):
                    sc = ev[m]
                    vv = kvp[r, pl.ds(128 + m * 16, 16)]
                    wp.at[r, pl.ds(m * 16, 16)][...] = vv * sc

        issue_idx(0)
        issue_idx(1)
        wait_idx(0)
        issue_gather(0)

        @pl.loop(0, nchw)
        def _(i):
            @pl.when(i + 1 < nchw)
            def _():
                wait_idx(i + 1)
                issue_gather(i + 1)
            wait_gather(i)

            @pl.when(i >= 2)
            def _():
                wait_scatter(i - 2)

            @pl.when(i + 2 < nchw)
            def _():
                issue_idx(i + 2)
            compute(i)
            issue_scatter(i)

        wait_scatter(nchw - 2)
        wait_scatter(nchw - 1)

        plsc.subcore_barrier()
        pltpu.sync_copy(acc_sh.at[pl.ds(sid * rpt, rpt)],
                        out_hbm.at[cid, pl.ds(sid * rpt, rpt)])

    return ek(kv, q, src, dst)


def _fin_body(ue_ref, wa_ref, ba_ref, hout_ref):
    n = hout_ref.shape[0]
    ue = ue_ref[0, :n] + ue_ref[1, :n]
    u = ue[:, :128]
    s = ue[:, 128:136]
    col = jax.lax.broadcasted_iota(jnp.int32, (8, 128), 1)
    row = jax.lax.broadcasted_iota(jnp.int32, (8, 128), 0)
    ind = jnp.where(col // 16 == row, 1.0, 0.0).astype(jnp.float32)
    s_exp = jax.lax.dot_general(s, ind, (((1,), (0,)), ((), ())), precision=_HI)
    agg = u / jnp.maximum(s_exp, 1e-30)
    hout = jax.lax.dot_general(agg, wa_ref[...], (((1,), (1,)), ((), ())),
                               precision=_HI) + ba_ref[...]
    hout_ref[...] = hout


def _finalize(ue, Wa, ba, n):
    return pl.pallas_call(
        _fin_body,
        out_shape=jax.ShapeDtypeStruct((n, 128), jnp.float32),
    )(ue, Wa, ba)


def _score_pass(hout, asrc, adst):
    te = asrc.shape[0]
    C = 64
    nchunks = te // C
    nb, rem = divmod(nchunks, NW)
    assert nchunks * C == te and C % L == 0 and nb >= 8

    mesh = plsc.VectorSubcoreMesh(core_axis_name="c", subcore_axis_name="s")

    @functools.partial(
        pl.kernel,
        out_type=jax.ShapeDtypeStruct((te,), jnp.float32),
        mesh=mesh,
        scratch_types=[
            pltpu.VMEM((8, C), jnp.int32),
            pltpu.VMEM((8, C), jnp.int32),
            pltpu.VMEM((4, C, 128), jnp.float32),
            pltpu.VMEM((4, C, 128), jnp.float32),
            pltpu.VMEM((4, C), jnp.float32),
            pltpu.SemaphoreType.DMA((8,)),
            pltpu.SemaphoreType.DMA((8,)),
            pltpu.SemaphoreType.DMA((4,)),
            pltpu.SemaphoreType.DMA((4,)),
            pltpu.SemaphoreType.DMA((4,)),
        ],
        compiler_params=_SC_PARAMS,
    )
    def sk(h_hbm, src_hbm, dst_hbm, out_hbm,
           sidx, didx, ab, bb, ob, si, sd, sa, sb, so):
        cid = lax.axis_index("c")
        sid = lax.axis_index("s")
        wid = sid * NC + cid
        nchw = nb + jnp.where(wid < rem, 1, 0)
        rows0 = jax.lax.iota(jnp.int32, L)

        def base_of(ch):
            return (wid + ch * NW) * C

        def issue_idx(ch):
            r = lax.rem(ch, 8)
            pltpu.async_copy(src_hbm.at[pl.ds(base_of(ch), C)], sidx.at[r],
                             si.at[r])
            pltpu.async_copy(dst_hbm.at[pl.ds(base_of(ch), C)], didx.at[r],
                             sd.at[r])

        def wait_idx(ch):
            r = lax.rem(ch, 8)
            pltpu.make_async_copy(src_hbm.at[pl.ds(base_of(ch), C)],
                                  sidx.at[r], si.at[r]).wait()
            pltpu.make_async_copy(dst_hbm.at[pl.ds(base_of(ch), C)],
                                  didx.at[r], sd.at[r]).wait()

        def issue_gather(ch):
            r = lax.rem(ch, 8)
            p = lax.rem(ch, 4)
            pltpu.async_copy(h_hbm.at[sidx.at[r]], ab.at[p], sa.at[p])
            pltpu.async_copy(h_hbm.at[didx.at[r]], bb.at[p], sb.at[p])

        def wait_gather(ch):
            r = lax.rem(ch, 8)
            p = lax.rem(ch, 4)
            pltpu.make_async_copy(h_hbm.at[sidx.at[r]], ab.at[p],
                                  sa.at[p]).wait()
            pltpu.make_async_copy(h_hbm.at[didx.at[r]], bb.at[p],
                                  sb.at[p]).wait()

        def issue_out(ch):
            p = lax.rem(ch, 4)
            pltpu.async_copy(ob.at[p], out_hbm.at[pl.ds(base_of(ch), C)],
                             so.at[p])

        def wait_out(ch):
            p = lax.rem(ch, 4)
            pltpu.make_async_copy(ob.at[p], out_hbm.at[pl.ds(base_of(ch), C)],
                                  so.at[p]).wait()

        def compute(ch):
            p = lax.rem(ch, 4)
            ap = ab.at[p]
            bp = bb.at[p]
            op = ob.at[p]
            for g in range(C // L):
                outv = jnp.zeros((L,), jnp.float32)
                for rr in range(L):
                    r = g * L + rr
                    acc = None
                    for m in range(8):
                        prod = (ap[r, pl.ds(m * 16, 16)]
                                * bp[r, pl.ds(m * 16, 16)])
                        acc = prod if acc is None else acc + prod
                    tot = jnp.sum(acc)
                    outv = jnp.where(rows0 == rr, tot, outv)
                op.at[pl.ds(g * L, L)][...] = outv

        for k in range(5):
            issue_idx(k)
        for k in range(3):
            wait_idx(k)
            issue_gather(k)

        @pl.loop(0, nchw)
        def _(i):
            @pl.when(i + 3 < nchw)
            def _():
                wait_idx(i + 3)
                issue_gather(i + 3)
            wait_gather(i)

            @pl.when(i >= 3)
            def _():
                wait_out(i - 3)

            @pl.when(i + 5 < nchw)
            def _():
                issue_idx(i + 5)
            compute(i)
            issue_out(i)

        wait_out(nchw - 3)
        wait_out(nchw - 2)
        wait_out(nchw - 1)

    return sk(hout, asrc, adst)


def kernel(h, edge_index, neg_edge_index, Wq, bq, Wk, bk, Wv, bv,
           Wmsg, bmsg, Wattn, battn, Wa, ba):
    e = edge_index.shape[1]
    src = edge_index[0].astype(jnp.int32)
    dst = edge_index[1].astype(jnp.int32)
    nsrc = neg_edge_index[0].astype(jnp.int32)
    ndst = neg_edge_index[1].astype(jnp.int32)

    kv, q = _project(h, Wq, bq.reshape(1, -1), Wk, bk.reshape(1, -1),
                     Wv, bv.reshape(1, -1), Wmsg, bmsg.reshape(1, -1),
                     Wattn, battn.reshape(1, -1))
    ue = _edge_pass(kv, q, src, dst)
    hout = _finalize(ue, Wa, ba.reshape(1, -1), h.shape[0])

    asrc = jnp.concatenate([src, nsrc])
    adst = jnp.concatenate([dst, ndst])
    sc = _score_pass(hout, asrc, adst)
    score = sc[:e].reshape(e, 1, 1)
    neg_score = sc[e:].reshape(e, 1, 1)
    return hout[:, None, :], score, neg_score
